# Initial kernel scaffold; baseline (speedup 1.0000x reference)
import jax
import jax.numpy as jnp
from jax.experimental import pallas as pl


def kernel(x, neighborhood):
    raise NotImplementedError("write your pallas kernel here")


# speedup vs baseline: 1.8998x; 1.8998x over previous
import functools

import jax
import jax.numpy as jnp
from jax import lax
from jax.experimental import pallas as pl
from jax.experimental.pallas import tpu as pltpu
from jax.experimental.pallas import tpu_sc as plsc

B = 2
F = 128
N_IN = 10000
N_OUT = 5000
K = 32
D = B * F
NW = 32
OUT_PAD = 5120
PER_W = OUT_PAD // NW
NB = 4
ROWS = NB * K
NBATCH = PER_W // NB
LANES = 16


_mesh = plsc.VectorSubcoreMesh(core_axis_name="c", subcore_axis_name="s")


@functools.partial(
    pl.kernel,
    out_type=jax.ShapeDtypeStruct((OUT_PAD, D), jnp.float32),
    mesh=_mesh,
    scratch_types=[
        pltpu.VMEM((PER_W * K,), jnp.int32),
        pltpu.VMEM((ROWS, D), jnp.float32),
        pltpu.VMEM((PER_W, D), jnp.float32),
        pltpu.SemaphoreType.DMA,
    ],
)
def _sc_gather_max(xt_hbm, idx_hbm, out_hbm, idx_v, buf, res_v, sem):
    wid = lax.axis_index("s") * 2 + lax.axis_index("c")
    base = wid * (PER_W * K)
    pltpu.sync_copy(idx_hbm.at[pl.ds(base, PER_W * K)], idx_v)

    def batch_body(j, _):
        pltpu.async_copy(
            xt_hbm.at[idx_v.at[pl.ds(j * ROWS, ROWS)]], buf, sem
        ).wait()

        def col_body(d, _):
            col = d * LANES
            for g in range(NB):
                r0 = g * K
                acc = buf[r0, pl.ds(col, LANES)]
                for r in range(1, K):
                    acc = jnp.maximum(acc, buf[r0 + r, pl.ds(col, LANES)])
                res_v[j * NB + g, pl.ds(col, LANES)] = acc
            return _

        lax.fori_loop(0, D // LANES, col_body, None)
        return _

    lax.fori_loop(0, NBATCH, batch_body, None)
    pltpu.sync_copy(res_v, out_hbm.at[pl.ds(wid * PER_W, PER_W)])


def kernel(x, neighborhood):
    xt = x.reshape(D, N_IN).T
    idx = jnp.zeros((OUT_PAD, K), jnp.int32)
    idx = idx.at[:N_OUT].set(neighborhood.astype(jnp.int32))
    out_t = _sc_gather_max(xt, idx.reshape(-1))
    return out_t[:N_OUT].T.reshape(B, F, N_OUT)


# speedup vs baseline: 2.1695x; 1.1420x over previous
import functools

import jax
import jax.numpy as jnp
from jax import lax
from jax.experimental import pallas as pl
from jax.experimental.pallas import tpu as pltpu
from jax.experimental.pallas import tpu_sc as plsc

B = 2
F = 128
N_IN = 10000
N_OUT = 5000
K = 32
D = B * F
NW = 32
OUT_PAD = 5120
PER_W = OUT_PAD // NW
NB = 4
ROWS = NB * K
NBATCH = PER_W // NB
LANES = 16


_mesh = plsc.VectorSubcoreMesh(core_axis_name="c", subcore_axis_name="s")


@functools.partial(
    pl.kernel,
    out_type=jax.ShapeDtypeStruct((OUT_PAD, D), jnp.float32),
    mesh=_mesh,
    scratch_types=[
        pltpu.VMEM((PER_W * K,), jnp.int32),
        pltpu.VMEM((ROWS, D), jnp.float32),
        pltpu.VMEM((ROWS, D), jnp.float32),
        pltpu.VMEM((PER_W, D), jnp.float32),
        pltpu.SemaphoreType.DMA,
        pltpu.SemaphoreType.DMA,
    ],
)
def _sc_gather_max(xt_hbm, idx_hbm, out_hbm, idx_v, buf_a, buf_b, res_v,
                   sem_a, sem_b):
    wid = lax.axis_index("s") * 2 + lax.axis_index("c")
    base = wid * (PER_W * K)
    pltpu.sync_copy(idx_hbm.at[pl.ds(base, PER_W * K)], idx_v)

    def gather_src(j):
        return xt_hbm.at[idx_v.at[pl.ds(j * ROWS, ROWS)]]

    def start(j, buf, sem):
        pltpu.async_copy(gather_src(j), buf, sem)

    def compute(j, buf):
        def col_body(d, _):
            col = d * LANES
            for g in range(NB):
                r0 = g * K
                acc = buf[r0, pl.ds(col, LANES)]
                for r in range(1, K):
                    acc = jnp.maximum(acc, buf[r0 + r, pl.ds(col, LANES)])
                res_v[j * NB + g, pl.ds(col, LANES)] = acc
            return _

        lax.fori_loop(0, D // LANES, col_body, None)

    start(0, buf_a, sem_a)
    start(1, buf_b, sem_b)

    def outer(j2, _):
        for b, (buf, sem) in enumerate(((buf_a, sem_a), (buf_b, sem_b))):
            j = j2 * 2 + b
            pltpu.make_async_copy(gather_src(j), buf, sem).wait()
            compute(j, buf)

            @pl.when(j + 2 < NBATCH)
            def _start_next():
                start(j + 2, buf, sem)
        return _

    lax.fori_loop(0, NBATCH // 2, outer, None)
    pltpu.sync_copy(res_v, out_hbm.at[pl.ds(wid * PER_W, PER_W)])


def kernel(x, neighborhood):
    xt = x.reshape(D, N_IN).T
    idx = jnp.zeros((OUT_PAD, K), jnp.int32)
    idx = idx.at[:N_OUT].set(neighborhood.astype(jnp.int32))
    out_t = _sc_gather_max(xt, idx.reshape(-1))
    return out_t[:N_OUT].T.reshape(B, F, N_OUT)


# speedup vs baseline: 2.1712x; 1.0008x over previous
import functools

import jax
import jax.numpy as jnp
from jax import lax
from jax.experimental import pallas as pl
from jax.experimental.pallas import tpu as pltpu
from jax.experimental.pallas import tpu_sc as plsc

B = 2
F = 128
N_IN = 10000
N_OUT = 5000
K = 32
D = B * F
NW = 32
OUT_PAD = 5120
PER_W = OUT_PAD // NW
NB = 4
ROWS = NB * K
NBATCH = PER_W // NB
LANES = 16


_mesh = plsc.VectorSubcoreMesh(core_axis_name="c", subcore_axis_name="s")


@functools.partial(
    pl.kernel,
    out_type=jax.ShapeDtypeStruct((OUT_PAD, D), jnp.float32),
    mesh=_mesh,
    scratch_types=[
        pltpu.VMEM((PER_W * K,), jnp.int32),
        pltpu.VMEM((ROWS, D), jnp.float32),
        pltpu.VMEM((ROWS, D), jnp.float32),
        pltpu.VMEM((PER_W, D), jnp.float32),
        pltpu.SemaphoreType.DMA,
        pltpu.SemaphoreType.DMA,
    ],
)
def _sc_gather_max(xt_hbm, idx_hbm, out_hbm, idx_v, buf_a, buf_b, res_v,
                   sem_a, sem_b):
    wid = lax.axis_index("s") * 2 + lax.axis_index("c")
    base = wid * (PER_W * K)
    pltpu.sync_copy(idx_hbm.at[pl.ds(base, PER_W * K)], idx_v)

    def gather_src(j):
        return xt_hbm.at[idx_v.at[pl.ds(j * ROWS, ROWS)]]

    def start(j, buf, sem):
        pltpu.async_copy(gather_src(j), buf, sem)

    def compute(j, buf):
        @plsc.parallel_loop(0, D // LANES)
        def col_body(d):
            col = d * LANES
            for g in range(NB):
                r0 = g * K
                vals = [
                    jnp.maximum(
                        buf[r0 + 2 * i, pl.ds(col, LANES)],
                        buf[r0 + 2 * i + 1, pl.ds(col, LANES)],
                    )
                    for i in range(K // 2)
                ]
                while len(vals) > 1:
                    vals = [
                        jnp.maximum(vals[2 * i], vals[2 * i + 1])
                        for i in range(len(vals) // 2)
                    ]
                res_v[j * NB + g, pl.ds(col, LANES)] = vals[0]

    start(0, buf_a, sem_a)
    start(1, buf_b, sem_b)

    def outer(j2, _):
        for b, (buf, sem) in enumerate(((buf_a, sem_a), (buf_b, sem_b))):
            j = j2 * 2 + b
            pltpu.make_async_copy(gather_src(j), buf, sem).wait()
            compute(j, buf)

            @pl.when(j + 2 < NBATCH)
            def _start_next():
                start(j + 2, buf, sem)
        return _

    lax.fori_loop(0, NBATCH // 2, outer, None)
    pltpu.sync_copy(res_v, out_hbm.at[pl.ds(wid * PER_W, PER_W)])


def kernel(x, neighborhood):
    xt = x.reshape(D, N_IN).T
    idx = jnp.zeros((OUT_PAD, K), jnp.int32)
    idx = idx.at[:N_OUT].set(neighborhood.astype(jnp.int32))
    out_t = _sc_gather_max(xt, idx.reshape(-1))
    return out_t[:N_OUT].T.reshape(B, F, N_OUT)


# speedup vs baseline: 5.2641x; 2.4245x over previous
import functools

import jax
import jax.numpy as jnp
from jax import lax
from jax.experimental import pallas as pl
from jax.experimental.pallas import tpu as pltpu
from jax.experimental.pallas import tpu_sc as plsc

B = 2
F = 128
N_IN = 10000
N_OUT = 5000
K = 32
D = B * F
NC = 2
NS = 16
DH = D // NC
OUT_PAD = 5120
PER_S = OUT_PAD // NS
NB = 4
ROWS = NB * K
NBATCH = PER_S // NB
NBUF = 4
N_IN_PAD = 10240
ROWS_PER_TILE = N_IN_PAD // NS
LANES = 16


_mesh = plsc.VectorSubcoreMesh(core_axis_name="c", subcore_axis_name="s")


@functools.partial(
    pl.kernel,
    out_type=jax.ShapeDtypeStruct((OUT_PAD, D), jnp.float32),
    mesh=_mesh,
    scratch_types=[
        pltpu.VMEM((PER_S * K,), jnp.int32),
        pltpu.VMEM((ROWS, DH), jnp.float32),
        pltpu.VMEM((ROWS, DH), jnp.float32),
        pltpu.VMEM((2 * NB, DH), jnp.float32),
        pltpu.VMEM((2 * NB, DH), jnp.float32),
        pltpu.VMEM_SHARED((N_IN_PAD, DH), jnp.float32),
        pltpu.SemaphoreType.DMA,
        pltpu.SemaphoreType.DMA,
        pltpu.SemaphoreType.DMA,
        pltpu.SemaphoreType.DMA,
    ],
)
def _sc_gather_max(xt_hbm, idx_hbm, out_hbm, idx_v, gbuf0, gbuf1, obuf0,
                   obuf1, table, gsem0, gsem1, osem0, osem1):
    c = lax.axis_index("c")
    s = lax.axis_index("s")

    pltpu.sync_copy(
        xt_hbm.at[c, pl.ds(s * ROWS_PER_TILE, ROWS_PER_TILE), :],
        table.at[pl.ds(s * ROWS_PER_TILE, ROWS_PER_TILE)],
    )
    pltpu.sync_copy(idx_hbm.at[pl.ds(s * (PER_S * K), PER_S * K)], idx_v)
    plsc.subcore_barrier()

    gbufs = (gbuf0, gbuf1)
    gsems = (gsem0, gsem1)
    obufs = (obuf0, obuf1)
    osems = (osem0, osem1)

    def gather_src(j):
        return table.at[idx_v.at[pl.ds(j * ROWS, ROWS)]]

    def out_dst(g):
        return out_hbm.at[
            pl.ds(s * PER_S + g * (2 * NB), 2 * NB), pl.ds(c * DH, DH)
        ]

    def compute(buf, obuf, q):
        @plsc.parallel_loop(0, DH // LANES)
        def col_body(d):
            col = d * LANES
            for g in range(NB):
                r0 = g * K
                vals = [
                    jnp.maximum(
                        buf[r0 + 2 * i, pl.ds(col, LANES)],
                        buf[r0 + 2 * i + 1, pl.ds(col, LANES)],
                    )
                    for i in range(K // 2)
                ]
                while len(vals) > 1:
                    vals = [
                        jnp.maximum(vals[2 * i], vals[2 * i + 1])
                        for i in range(len(vals) // 2)
                    ]
                obuf[q * NB + g, pl.ds(col, LANES)] = vals[0]

    pltpu.async_copy(gather_src(0), gbufs[0], gsems[0])
    pltpu.async_copy(gather_src(1), gbufs[1], gsems[1])

    def outer(g2, _):
        for gg in range(2):
            g = g2 * 2 + gg
            obuf, osem = obufs[gg], osems[gg]

            @pl.when(g >= 2)
            def _drain_out():
                pltpu.make_async_copy(obuf, out_dst(g - 2), osem).wait()

            for q in range(2):
                j = g * 2 + q
                pltpu.make_async_copy(gather_src(j), gbufs[q], gsems[q]).wait()
                compute(gbufs[q], obuf, q)

                @pl.when(j + 2 < NBATCH)
                def _start_next():
                    pltpu.async_copy(gather_src(j + 2), gbufs[q], gsems[q])

            pltpu.async_copy(obuf, out_dst(g), osem)
        return _

    NGROUP = NBATCH // 2
    lax.fori_loop(0, NGROUP // 2, outer, None)
    for gg in range(2):
        pltpu.make_async_copy(
            obufs[gg], out_dst(NGROUP - 2 + gg), osems[gg]
        ).wait()


def kernel(x, neighborhood):
    xt = x.reshape(D, N_IN).T.reshape(N_IN, NC, DH).transpose(1, 0, 2)
    xt = jnp.pad(xt, ((0, 0), (0, N_IN_PAD - N_IN), (0, 0)))
    idx = jnp.zeros((OUT_PAD, K), jnp.int32)
    idx = idx.at[:N_OUT].set(neighborhood.astype(jnp.int32))
    out_t = _sc_gather_max(xt, idx.reshape(-1))
    return out_t[:N_OUT].T.reshape(B, F, N_OUT)
